# baseline (speedup 1.0000x reference)
import jax
import jax.numpy as jnp
from jax.experimental import pallas as pl
from jax.experimental.pallas import tpu as pltpu

BATCH = 8
N_TILES = 4
N_TOKENS = 1025
EMBED_DIM = 1280
MAX_NUM_TILES = 4

ECH = 2
EW = EMBED_DIM // ECH
ITEMS = BATCH * N_TILES * ECH
NSLOTS = 8


def _coords(k):
    bt = k // ECH
    e = k - bt * ECH
    b = bt // N_TILES
    t = bt - b * N_TILES
    as_i32 = lambda v: jnp.asarray(v, dtype=jnp.int32)
    return as_i32(b), as_i32(t), as_i32(e)


def _in_copy(k, x_ref, in_buf, in_sem):
    b, t, e = _coords(k)
    return pltpu.make_async_copy(
        x_ref.at[b, t, :, pl.ds(e * EW, EW)], in_buf, in_sem)


def _out_copy(k, out_ref, out_buf, out_sem):
    b, t, e = _coords(k)
    return pltpu.make_async_copy(
        out_buf, out_ref.at[b, t, :, pl.ds(e * EW, EW)], out_sem)


def _body(ar_ref, gate_ref, x_ref, emb_ref, out_ref, *scratch):
    in_bufs = scratch[0:NSLOTS]
    out_bufs = scratch[NSLOTS:2 * NSLOTS]
    in_sems = scratch[2 * NSLOTS:3 * NSLOTS]
    out_sems = scratch[3 * NSLOTS:4 * NSLOTS]

    for s in range(NSLOTS):
        _in_copy(s, x_ref, in_bufs[s], in_sems[s]).start()

    gate_t = jnp.tanh(gate_ref[0])

    def make_step(s):
        def step(k):
            _in_copy(k, x_ref, in_bufs[s], in_sems[s]).wait()

            b, t, e = _coords(k)
            h = ar_ref[2 * b]
            w = ar_ref[2 * b + 1]
            valid = t < h * w
            w_safe = jnp.maximum(w, 1)
            i = jnp.where(valid, t // w_safe, 0)
            j = jnp.where(valid, t % w_safe, 0)
            row = emb_ref[i, j, 0, pl.ds(e * EW, EW)]
            coef = jnp.where(valid, gate_t, 0.0)

            @pl.when(k >= NSLOTS)
            def _():
                _out_copy(k - NSLOTS, out_ref, out_bufs[s], out_sems[s]).wait()

            out_bufs[s][...] = in_bufs[s][...] + (coef * row)[None, :]

            _out_copy(k, out_ref, out_bufs[s], out_sems[s]).start()

            @pl.when(k + NSLOTS < ITEMS)
            def _():
                _in_copy(k + NSLOTS, x_ref, in_bufs[s], in_sems[s]).start()
        return step

    steps = [make_step(s) for s in range(NSLOTS)]

    def loop(r, carry):
        base = r * NSLOTS
        for s in range(NSLOTS):
            steps[s](base + jnp.int32(s))
        return carry

    jax.lax.fori_loop(jnp.int32(0), jnp.int32(ITEMS // NSLOTS), loop,
                      jnp.int32(0))

    for s in range(NSLOTS):
        k = ITEMS - NSLOTS + s
        _out_copy(k, out_ref, out_bufs[s], out_sems[s]).wait()


def kernel(x, aspect_ratio, embedding, gate):
    ar = aspect_ratio.astype(jnp.int32).reshape(-1)

    scratch_shapes = (
        [pltpu.VMEM((N_TOKENS, EW), jnp.float32) for _ in range(NSLOTS)]
        + [pltpu.VMEM((N_TOKENS, EW), jnp.float32) for _ in range(NSLOTS)]
        + [pltpu.SemaphoreType.DMA for _ in range(2 * NSLOTS)]
    )

    grid_spec = pltpu.PrefetchScalarGridSpec(
        num_scalar_prefetch=2,
        grid=(1, 1),
        in_specs=[
            pl.BlockSpec(memory_space=pl.ANY),
            pl.BlockSpec((MAX_NUM_TILES, MAX_NUM_TILES, 1, EMBED_DIM),
                         lambda z0, z1, ar, g: (z0, z1, z0, z1)),
        ],
        out_specs=pl.BlockSpec(memory_space=pl.ANY),
        scratch_shapes=scratch_shapes,
    )

    out = pl.pallas_call(
        _body,
        grid_spec=grid_spec,
        out_shape=jax.ShapeDtypeStruct(x.shape, x.dtype),
    )(ar, gate.astype(jnp.float32), x, embedding)
    return out


# speedup vs baseline: 4.1311x; 4.1311x over previous
import jax
import jax.numpy as jnp
from jax.experimental import pallas as pl
from jax.experimental.pallas import tpu as pltpu

BATCH = 8
N_TILES = 4
N_TOKENS = 1025
EMBED_DIM = 1280
MAX_NUM_TILES = 4

NCH = 5
CH = N_TOKENS // NCH


def _body(ar_ref, gate_ref, x_ref, emb_ref, out_ref):
    b = pl.program_id(0)
    h = ar_ref[2 * b]
    w = ar_ref[2 * b + 1]
    n = h * w
    w_safe = jnp.maximum(w, 1)
    gate_t = jnp.tanh(gate_ref[0])

    rows = []
    for t in range(N_TILES):
        valid = t < n
        i = jnp.where(valid, t // w_safe, 0)
        j = jnp.where(valid, t % w_safe, 0)
        row = emb_ref[i, j]
        coef = jnp.where(valid, gate_t, 0.0)
        rows.append(coef * row)
    add = jnp.concatenate(rows, axis=0)

    out_ref[...] = x_ref[...] + add[None, None, :, :]


def kernel(x, aspect_ratio, embedding, gate):
    ar = aspect_ratio.astype(jnp.int32).reshape(-1)
    xt = jnp.transpose(x, (0, 2, 1, 3))

    grid_spec = pltpu.PrefetchScalarGridSpec(
        num_scalar_prefetch=2,
        grid=(BATCH, NCH, 1, 1),
        in_specs=[
            pl.BlockSpec((1, CH, N_TILES, EMBED_DIM),
                         lambda b, c, z0, z1, ar, g: (b, c, z0, z1)),
            pl.BlockSpec((MAX_NUM_TILES, MAX_NUM_TILES, 1, EMBED_DIM),
                         lambda b, c, z0, z1, ar, g: (z0, z1, z0, z1)),
        ],
        out_specs=pl.BlockSpec((1, CH, N_TILES, EMBED_DIM),
                               lambda b, c, z0, z1, ar, g: (b, c, z0, z1)),
    )

    out = pl.pallas_call(
        _body,
        grid_spec=grid_spec,
        out_shape=jax.ShapeDtypeStruct(xt.shape, xt.dtype),
    )(ar, gate.astype(jnp.float32), xt, embedding)
    return jnp.transpose(out, (0, 2, 1, 3))
